# baseline (speedup 1.0000x reference)
import functools

import jax
import jax.numpy as jnp
from jax import lax
from jax.experimental import pallas as pl
from jax.experimental.pallas import tpu as pltpu
from jax.experimental.pallas import tpu_sc as plsc

_L = 2
_N = 10000
_E = 160000
_NPAD = 10112
_NT = 32
_EPT = 5184
_EPAD = _NT * _EPT
_K = 32
_NCH = _EPT // _K
_GRP = 6
_RPS = _NPAD // 16


def _matmul_body(x_ref, w_ref, o_ref):
    o_ref[...] = jnp.dot(x_ref[...], w_ref[...],
                         preferred_element_type=jnp.float32)


def _matmul(x, w, bn):
    n, k = x.shape
    m = w.shape[1]
    return pl.pallas_call(
        _matmul_body,
        grid=(n // bn,),
        in_specs=[pl.BlockSpec((bn, k), lambda i: (i, 0)),
                  pl.BlockSpec((k, m), lambda i: (0, 0))],
        out_specs=pl.BlockSpec((bn, m), lambda i: (i, 0)),
        out_shape=jax.ShapeDtypeStruct((n, m), jnp.float32),
    )(x, w)


def _onehot_mm_body(x_ref, t_ref, o_ref):
    x = x_ref[...]
    n, d = x.shape
    m = jnp.max(x, axis=1, keepdims=True)
    iota = lax.broadcasted_iota(jnp.int32, (n, d), 1)
    idx = jnp.min(jnp.where(x == m, iota, d), axis=1, keepdims=True)
    oh = (iota == idx).astype(jnp.float32)
    o_ref[...] = jnp.dot(oh, t_ref[...], preferred_element_type=jnp.float32)


def _onehot_mm(x, table, bn):
    n, d = x.shape
    m = table.shape[1]
    return pl.pallas_call(
        _onehot_mm_body,
        grid=(n // bn,),
        in_specs=[pl.BlockSpec((bn, d), lambda i: (i, 0)),
                  pl.BlockSpec((d, m), lambda i: (0, 0))],
        out_specs=pl.BlockSpec((bn, m), lambda i: (i, 0)),
        out_shape=jax.ShapeDtypeStruct((n, m), jnp.float32),
    )(x, table)


def _pool_body(x_ref, acc_ref, b_ref, ids_ref, s_ref, c_ref):
    i = pl.program_id(0)
    xf = x_ref[...] + acc_ref[0] + acc_ref[1] + b_ref[...]
    n = xf.shape[0]
    oh_t = (lax.broadcasted_iota(jnp.int32, (64, n), 0)
            == ids_ref[0]).astype(jnp.float32)

    @pl.when(i == 0)
    def _():
        s_ref[...] = jnp.zeros_like(s_ref)
        c_ref[...] = jnp.zeros_like(c_ref)

    s_ref[...] += jnp.dot(oh_t, xf, preferred_element_type=jnp.float32)
    c_ref[...] += jnp.dot(oh_t, jnp.ones((n, 128), jnp.float32),
                          preferred_element_type=jnp.float32)


def _pool(x, acc2, bias, ids):
    bn = 400
    s, cnt = pl.pallas_call(
        _pool_body,
        grid=(_N // bn,),
        in_specs=[pl.BlockSpec((bn, 128), lambda i: (i, 0)),
                  pl.BlockSpec((2, bn, 128), lambda i: (0, i, 0)),
                  pl.BlockSpec((1, 128), lambda i: (0, 0)),
                  pl.BlockSpec((1, 1, bn), lambda i: (i, 0, 0))],
        out_specs=[pl.BlockSpec((64, 128), lambda i: (0, 0)),
                   pl.BlockSpec((64, 128), lambda i: (0, 0))],
        out_shape=[jax.ShapeDtypeStruct((64, 128), jnp.float32),
                   jax.ShapeDtypeStruct((64, 128), jnp.float32)],
    )(x, acc2.reshape(2, _NPAD, 128), bias.reshape(1, 128),
      ids.reshape(_N // bn, 1, bn))
    return s / jnp.maximum(cnt, 1.0)


def _proj0_body(x_ref, w_ref, rp_ref, cp_ref, rg_ref, cg_ref):
    rc = jnp.dot(x_ref[...], w_ref[...], preferred_element_type=jnp.float32)
    rp_ref[...] = rc[:, 0:128]
    cp_ref[...] = rc[:, 128:256]
    rg_ref[...] = rc[:, 256:288]
    cg_ref[...] = rc[:, 288:320]


def _proj0(x, w):
    bn = 632
    return pl.pallas_call(
        _proj0_body,
        grid=(_NPAD // bn,),
        in_specs=[pl.BlockSpec((bn, 128), lambda i: (i, 0)),
                  pl.BlockSpec((128, 320), lambda i: (0, 0))],
        out_specs=[pl.BlockSpec((bn, 128), lambda i: (i, 0)),
                   pl.BlockSpec((bn, 128), lambda i: (i, 0)),
                   pl.BlockSpec((bn, 32), lambda i: (i, 0)),
                   pl.BlockSpec((bn, 32), lambda i: (i, 0))],
        out_shape=[jax.ShapeDtypeStruct((_NPAD, 128), jnp.float32),
                   jax.ShapeDtypeStruct((_NPAD, 128), jnp.float32),
                   jax.ShapeDtypeStruct((_NPAD, 32), jnp.float32),
                   jax.ShapeDtypeStruct((_NPAD, 32), jnp.float32)],
    )(x, w)


def _proj_body(x_ref, acc_ref, b_ref, w_ref, xn_ref, rp_ref, cp_ref,
               rg_ref, cg_ref):
    xn = (x_ref[...] + acc_ref[0] + acc_ref[1] + b_ref[...])
    xn_ref[...] = xn
    rc = jnp.dot(xn, w_ref[...], preferred_element_type=jnp.float32)
    rp_ref[...] = rc[:, 0:128]
    cp_ref[...] = rc[:, 128:256]
    rg_ref[...] = rc[:, 256:288]
    cg_ref[...] = rc[:, 288:320]


def _proj(x, acc2, bias, w):
    bn = 632
    return pl.pallas_call(
        _proj_body,
        grid=(_NPAD // bn,),
        in_specs=[pl.BlockSpec((bn, 128), lambda i: (i, 0)),
                  pl.BlockSpec((2, bn, 128), lambda i: (0, i, 0)),
                  pl.BlockSpec((1, 128), lambda i: (0, 0)),
                  pl.BlockSpec((128, 320), lambda i: (0, 0))],
        out_specs=[pl.BlockSpec((bn, 128), lambda i: (i, 0)),
                   pl.BlockSpec((bn, 128), lambda i: (i, 0)),
                   pl.BlockSpec((bn, 128), lambda i: (i, 0)),
                   pl.BlockSpec((bn, 32), lambda i: (i, 0)),
                   pl.BlockSpec((bn, 32), lambda i: (i, 0))],
        out_shape=[jax.ShapeDtypeStruct((_NPAD, 128), jnp.float32),
                   jax.ShapeDtypeStruct((_NPAD, 128), jnp.float32),
                   jax.ShapeDtypeStruct((_NPAD, 128), jnp.float32),
                   jax.ShapeDtypeStruct((_NPAD, 32), jnp.float32),
                   jax.ShapeDtypeStruct((_NPAD, 32), jnp.float32)],
    )(x, acc2.reshape(2, _NPAD, 128), bias.reshape(1, 128), w)


def _sc_body(Rph, Cph, Rgh, Cgh, ri2h, ci2h, ein_h, eout_h, wih, woh, zh,
             acc_h, a2_h, b2_h,
             acc_s, ridx_t, cidx_t,
             rp_b, cp_b, rg_b, cg_b, ein_b, eout_b, av_b, bv_b,
             gi_b, go_b, wiv, wov, sem_g, sem_s, sem_w):
    c = lax.axis_index("c")
    s = lax.axis_index("s")
    w = c * 16 + s
    pltpu.sync_copy(zh.at[pl.ds(s * _RPS, _RPS)],
                    acc_s.at[pl.ds(s * _RPS, _RPS)])
    pltpu.sync_copy(wih, wiv)
    pltpu.sync_copy(woh, wov)
    pltpu.sync_copy(ri2h.at[pl.ds(w * _NCH, _NCH)], ridx_t)
    pltpu.sync_copy(ci2h.at[pl.ds(w * _NCH, _NCH)], cidx_t)
    plsc.subcore_barrier()
    base = w * _EPT
    wir = wiv[...]
    wor = wov[...]

    def gathers(j, ps, sl):
        off = base + j * _K
        return (
            (Rph.at[ridx_t.at[j]], rp_b[ps], sem_g[ps]),
            (Cph.at[cidx_t.at[j]], cp_b[ps], sem_g[ps]),
            (Rgh.at[ridx_t.at[j]], rg_b[sl], sem_g[ps]),
            (Cgh.at[cidx_t.at[j]], cg_b[sl], sem_g[ps]),
            (ein_h.at[pl.ds(off, _K)], ein_b[sl], sem_g[ps]),
            (eout_h.at[pl.ds(off, _K)], eout_b[sl], sem_g[ps]),
        )

    def scatters(j, ps):
        return (
            (rp_b[ps], acc_s.at[cidx_t.at[j]], sem_s[ps]),
            (cp_b[ps], acc_s.at[ridx_t.at[j]], sem_s[ps]),
        )

    def writes(j, sl):
        off = base + j * _K
        return (
            (av_b[sl], a2_h.at[pl.ds(off, _K)], sem_w[sl]),
            (bv_b[sl], b2_h.at[pl.ds(off, _K)], sem_w[sl]),
        )

    def start_all(triples, add=False):
        for src, dst, sem in triples:
            pltpu.async_copy(src, dst, sem, add=add)

    def wait_all(triples):
        for src, dst, sem in triples:
            pltpu.make_async_copy(src, dst, sem).wait()

    def compute(ps, sl):
        rp, cp = rp_b[ps], cp_b[ps]
        rg, cg = rg_b[sl], cg_b[sl]
        ein, eout = ein_b[sl], eout_b[sl]
        av, bv = av_b[sl], bv_b[sl]

        def gates(t, carry):
            for u in range(4):
                e = t * 4 + u
                a = rg[e, pl.ds(0, 16)] + ein[e, :] + cg[e, pl.ds(0, 16)]
                b = rg[e, pl.ds(16, 16)] + eout[e, :] + cg[e, pl.ds(16, 16)]
                av[e, :] = a
                bv[e, :] = b
                zi = jnp.broadcast_to(jnp.sum(a * wir, axis=0), (16,))
                zo = jnp.broadcast_to(jnp.sum(b * wor, axis=0), (16,))
                gi_b[e, :] = 1.0 / (1.0 + jnp.exp(-zi))
                go_b[e, :] = 1.0 / (1.0 + jnp.exp(-zo))
            return carry

        lax.fori_loop(0, _K // 4, gates, 0)

        def scale(t, carry):
            for u in range(2):
                e = t * 2 + u
                gi = gi_b[e, :]
                go = go_b[e, :]
                for q in range(8):
                    rp[e, pl.ds(q * 16, 16)] = gi * rp[e, pl.ds(q * 16, 16)]
                    cp[e, pl.ds(q * 16, 16)] = go * cp[e, pl.ds(q * 16, 16)]
            return carry

        lax.fori_loop(0, _K // 2, scale, 0)

    start_all(gathers(0, 0, 0))

    def ring(g, carry):
        for u in range(_GRP):
            j = g * _GRP + u
            ps, sl = u % 3, u % 2
            psm2 = (u - 2) % 3
            psp1, slp1 = (u + 1) % 3, (u + 1) % 2

            @pl.when(j >= 2)
            def _():
                wait_all(scatters(j - 2, psm2))
                wait_all(writes(j - 2, sl))

            @pl.when(j + 1 < _NCH)
            def _():
                start_all(gathers(j + 1, psp1, slp1))

            wait_all(gathers(j, ps, sl))
            compute(ps, sl)
            start_all(scatters(j, ps), add=True)
            start_all(writes(j, sl))
        return carry

    lax.fori_loop(0, _NCH // _GRP, ring, 0)
    for j in (_NCH - 2, _NCH - 1):
        u = j % _GRP
        wait_all(scatters(j, u % 3))
        wait_all(writes(j, u % 2))
    plsc.subcore_barrier()
    pltpu.sync_copy(acc_s.at[pl.ds(s * _RPS, _RPS)],
                    acc_h.at[pl.ds(c * _NPAD + s * _RPS, _RPS)])


def _sc_sparse(Rp, Cp, Rgt, Cgt, ridx2, cidx2, eawin, eawout, wi1, wo1,
               zeros_nd):
    mesh = plsc.VectorSubcoreMesh(core_axis_name="c", subcore_axis_name="s")
    f = pl.kernel(
        _sc_body,
        out_type=(jax.ShapeDtypeStruct((2 * _NPAD, 128), jnp.float32),
                  jax.ShapeDtypeStruct((_EPAD, 16), jnp.float32),
                  jax.ShapeDtypeStruct((_EPAD, 16), jnp.float32)),
        mesh=mesh,
        compiler_params=pltpu.CompilerParams(needs_layout_passes=False,
                                             use_tc_tiling_on_sc=False),
        scratch_types=[
            pltpu.VMEM_SHARED((_NPAD, 128), jnp.float32),
            pltpu.VMEM((_NCH, _K), jnp.int32),
            pltpu.VMEM((_NCH, _K), jnp.int32),
            [pltpu.VMEM((_K, 128), jnp.float32) for _ in range(3)],
            [pltpu.VMEM((_K, 128), jnp.float32) for _ in range(3)],
            [pltpu.VMEM((_K, 32), jnp.float32) for _ in range(2)],
            [pltpu.VMEM((_K, 32), jnp.float32) for _ in range(2)],
            [pltpu.VMEM((_K, 16), jnp.float32) for _ in range(2)],
            [pltpu.VMEM((_K, 16), jnp.float32) for _ in range(2)],
            [pltpu.VMEM((_K, 16), jnp.float32) for _ in range(2)],
            [pltpu.VMEM((_K, 16), jnp.float32) for _ in range(2)],
            pltpu.VMEM((_K, 16), jnp.float32),
            pltpu.VMEM((_K, 16), jnp.float32),
            pltpu.VMEM((16,), jnp.float32),
            pltpu.VMEM((16,), jnp.float32),
            [pltpu.SemaphoreType.DMA for _ in range(3)],
            [pltpu.SemaphoreType.DMA for _ in range(3)],
            [pltpu.SemaphoreType.DMA for _ in range(2)],
        ],
    )
    return f(Rp, Cp, Rgt, Cgt, ridx2, cidx2, eawin, eawout, wi1, wo1,
             zeros_nd)


def kernel(x_s, x_t, edge_attr_s, edge_attr_t, edge_index_s, edge_index_t,
           x_s_batch, x_t_batch, y, node_emb, edge_emb, W_in_0, W_in_1,
           W_out_0, W_out_1, gin_W, gin_b, gout_W, gout_b, cls_W1, cls_b1,
           cls_W2, cls_b2):
    zeros_nd = jnp.zeros((_NPAD, 128), jnp.float32)

    def pad_e(a):
        return jnp.concatenate(
            [a, jnp.zeros((_EPAD - _E, 16), jnp.float32)], axis=0)

    def pad_idx(ix):
        return jnp.concatenate(
            [ix.astype(jnp.int32),
             jnp.full((_EPAD - _E,), _N, jnp.int32)],
            axis=0).reshape(_NT * _NCH, _K)

    Tin0 = edge_emb @ W_in_0[0][128:144]
    Tout0 = edge_emb @ W_out_0[0][128:144]

    def dag(x_raw, ea, ridx2, cidx2, batch_ids):
        x0 = jnp.concatenate(
            [_onehot_mm(x_raw, node_emb, 400),
             jnp.zeros((_NPAD - _N, 128), jnp.float32)], axis=0)
        eawin = pad_e(_onehot_mm(ea[:_E], Tin0, 1000))
        eawout = pad_e(_onehot_mm(ea[_E:], Tout0, 1000))
        x = x0
        acc2 = bias = a2 = b2 = None
        for i in range(_L):
            W = jnp.concatenate(
                [gin_W[i], gout_W[i], W_in_0[i][:128], W_out_0[i][144:],
                 W_in_0[i][144:], W_out_0[i][:128]], axis=1)
            if i == 0:
                Rp, Cp, Rgt, Cgt = _proj0(x, W)
            else:
                x, Rp, Cp, Rgt, Cgt = _proj(x, acc2, bias, W)
                eawin = _matmul(a2, W_in_0[i][128:144], 1024)
                eawout = _matmul(b2, W_out_0[i][128:144], 1024)
            acc2, a2, b2 = _sc_sparse(
                Rp, Cp, Rgt, Cgt, ridx2, cidx2, eawin, eawout,
                W_in_1[i].reshape(16), W_out_1[i].reshape(16), zeros_nd)
            bias = gin_b[i] + gout_b[i]
        return _pool(x, acc2, bias, batch_ids)

    g_s = dag(x_s, edge_attr_s, pad_idx(edge_index_s[0]),
              pad_idx(edge_index_s[1]), x_s_batch)
    g_t = dag(x_t, edge_attr_t, pad_idx(edge_index_t[0]),
              pad_idx(edge_index_t[1]), x_t_batch)

    B = y.shape[0]
    z = jnp.concatenate([g_s, g_t], axis=1)
    h = jax.nn.relu(z @ cls_W1 + cls_b1)
    pred = h @ cls_W2 + cls_b2
    logp = jax.nn.log_softmax(pred, axis=1)
    loss = -jnp.mean(logp[jnp.arange(B), y])
    return loss


# speedup vs baseline: 1.1510x; 1.1510x over previous
import functools

import jax
import jax.numpy as jnp
from jax import lax
from jax.experimental import pallas as pl
from jax.experimental.pallas import tpu as pltpu
from jax.experimental.pallas import tpu_sc as plsc

_L = 2
_N = 10000
_E = 160000
_NPAD = 10112
_NT = 32
_EPT = 5120
_EPAD = _NT * _EPT
_K = 32
_NCH = _EPT // _K
_GRP = 4
_RPS = _NPAD // 16


def _matmul_body(x_ref, w_ref, o_ref):
    o_ref[...] = jnp.dot(x_ref[...], w_ref[...],
                         preferred_element_type=jnp.float32)


def _matmul(x, w, bn):
    n, k = x.shape
    m = w.shape[1]
    return pl.pallas_call(
        _matmul_body,
        grid=(n // bn,),
        in_specs=[pl.BlockSpec((bn, k), lambda i: (i, 0)),
                  pl.BlockSpec((k, m), lambda i: (0, 0))],
        out_specs=pl.BlockSpec((bn, m), lambda i: (i, 0)),
        out_shape=jax.ShapeDtypeStruct((n, m), jnp.float32),
    )(x, w)


def _onehot_mm_body(x_ref, t_ref, o_ref):
    x = x_ref[...]
    n, d = x.shape
    m = jnp.max(x, axis=1, keepdims=True)
    iota = lax.broadcasted_iota(jnp.int32, (n, d), 1)
    idx = jnp.min(jnp.where(x == m, iota, d), axis=1, keepdims=True)
    oh = (iota == idx).astype(jnp.float32)
    o_ref[...] = jnp.dot(oh, t_ref[...], preferred_element_type=jnp.float32)


def _onehot_mm(x, table, bn):
    n, d = x.shape
    m = table.shape[1]
    return pl.pallas_call(
        _onehot_mm_body,
        grid=(n // bn,),
        in_specs=[pl.BlockSpec((bn, d), lambda i: (i, 0)),
                  pl.BlockSpec((d, m), lambda i: (0, 0))],
        out_specs=pl.BlockSpec((bn, m), lambda i: (i, 0)),
        out_shape=jax.ShapeDtypeStruct((n, m), jnp.float32),
    )(x, table)


def _pool_body(x_ref, acc_ref, b_ref, ids_ref, s_ref, c_ref):
    i = pl.program_id(0)
    xf = x_ref[...] + acc_ref[0] + acc_ref[1] + b_ref[...]
    n = xf.shape[0]
    oh_t = (lax.broadcasted_iota(jnp.int32, (64, n), 0)
            == ids_ref[0]).astype(jnp.float32)

    @pl.when(i == 0)
    def _():
        s_ref[...] = jnp.zeros_like(s_ref)
        c_ref[...] = jnp.zeros_like(c_ref)

    s_ref[...] += jnp.dot(oh_t, xf, preferred_element_type=jnp.float32)
    c_ref[...] += jnp.dot(oh_t, jnp.ones((n, 128), jnp.float32),
                          preferred_element_type=jnp.float32)


def _pool(x, acc2, bias, ids):
    bn = 400
    s, cnt = pl.pallas_call(
        _pool_body,
        grid=(_N // bn,),
        in_specs=[pl.BlockSpec((bn, 128), lambda i: (i, 0)),
                  pl.BlockSpec((2, bn, 128), lambda i: (0, i, 0)),
                  pl.BlockSpec((1, 128), lambda i: (0, 0)),
                  pl.BlockSpec((1, 1, bn), lambda i: (i, 0, 0))],
        out_specs=[pl.BlockSpec((64, 128), lambda i: (0, 0)),
                   pl.BlockSpec((64, 128), lambda i: (0, 0))],
        out_shape=[jax.ShapeDtypeStruct((64, 128), jnp.float32),
                   jax.ShapeDtypeStruct((64, 128), jnp.float32)],
    )(x, acc2.reshape(2, _NPAD, 128), bias.reshape(1, 128),
      ids.reshape(_N // bn, 1, bn))
    return s / jnp.maximum(cnt, 1.0)


def _proj0_body(x_ref, w_ref, rp_ref, cp_ref, rg_ref, cg_ref):
    rc = jnp.dot(x_ref[...], w_ref[...], preferred_element_type=jnp.float32)
    rp_ref[...] = rc[:, 0:128]
    cp_ref[...] = rc[:, 128:256]
    rg_ref[...] = rc[:, 256:288]
    cg_ref[...] = rc[:, 288:320]


def _proj0(x, w):
    bn = 632
    return pl.pallas_call(
        _proj0_body,
        grid=(_NPAD // bn,),
        in_specs=[pl.BlockSpec((bn, 128), lambda i: (i, 0)),
                  pl.BlockSpec((128, 320), lambda i: (0, 0))],
        out_specs=[pl.BlockSpec((bn, 128), lambda i: (i, 0)),
                   pl.BlockSpec((bn, 128), lambda i: (i, 0)),
                   pl.BlockSpec((bn, 32), lambda i: (i, 0)),
                   pl.BlockSpec((bn, 32), lambda i: (i, 0))],
        out_shape=[jax.ShapeDtypeStruct((_NPAD, 128), jnp.float32),
                   jax.ShapeDtypeStruct((_NPAD, 128), jnp.float32),
                   jax.ShapeDtypeStruct((_NPAD, 32), jnp.float32),
                   jax.ShapeDtypeStruct((_NPAD, 32), jnp.float32)],
    )(x, w)


def _proj_body(x_ref, acc_ref, b_ref, w_ref, xn_ref, rp_ref, cp_ref,
               rg_ref, cg_ref):
    xn = (x_ref[...] + acc_ref[0] + acc_ref[1] + b_ref[...])
    xn_ref[...] = xn
    rc = jnp.dot(xn, w_ref[...], preferred_element_type=jnp.float32)
    rp_ref[...] = rc[:, 0:128]
    cp_ref[...] = rc[:, 128:256]
    rg_ref[...] = rc[:, 256:288]
    cg_ref[...] = rc[:, 288:320]


def _proj(x, acc2, bias, w):
    bn = 632
    return pl.pallas_call(
        _proj_body,
        grid=(_NPAD // bn,),
        in_specs=[pl.BlockSpec((bn, 128), lambda i: (i, 0)),
                  pl.BlockSpec((2, bn, 128), lambda i: (0, i, 0)),
                  pl.BlockSpec((1, 128), lambda i: (0, 0)),
                  pl.BlockSpec((128, 320), lambda i: (0, 0))],
        out_specs=[pl.BlockSpec((bn, 128), lambda i: (i, 0)),
                   pl.BlockSpec((bn, 128), lambda i: (i, 0)),
                   pl.BlockSpec((bn, 128), lambda i: (i, 0)),
                   pl.BlockSpec((bn, 32), lambda i: (i, 0)),
                   pl.BlockSpec((bn, 32), lambda i: (i, 0))],
        out_shape=[jax.ShapeDtypeStruct((_NPAD, 128), jnp.float32),
                   jax.ShapeDtypeStruct((_NPAD, 128), jnp.float32),
                   jax.ShapeDtypeStruct((_NPAD, 128), jnp.float32),
                   jax.ShapeDtypeStruct((_NPAD, 32), jnp.float32),
                   jax.ShapeDtypeStruct((_NPAD, 32), jnp.float32)],
    )(x, acc2.reshape(2, _NPAD, 128), bias.reshape(1, 128), w)


def _sc_body(Rph, Cph, Rgh, Cgh, rih, cih, ein_h, eout_h, wih, woh, zh,
             acc_h, ab2_h,
             acc_s, rp_b, cp_b, rg_b, cg_b, ein_b, eout_b,
             ri_b, ci_b, sri_b, sci_b, wiv, wov,
             sem_i, sem_g, sem_s, sem_w):
    c = lax.axis_index("c")
    s = lax.axis_index("s")
    w = c * 16 + s
    pltpu.sync_copy(zh.at[pl.ds(s * _RPS, _RPS)],
                    acc_s.at[pl.ds(s * _RPS, _RPS)])
    pltpu.sync_copy(wih, wiv)
    pltpu.sync_copy(woh, wov)
    plsc.subcore_barrier()
    base = w * _EPT
    wir = wiv[...]
    wor = wov[...]

    def istage(j, k):
        off = base + j * _K
        return ((rih.at[pl.ds(off, _K)], ri_b[k], sem_i[k]),
                (cih.at[pl.ds(off, _K)], ci_b[k], sem_i[k]))

    def gstage(j, k):
        off = base + j * _K
        return ((Rph.at[ri_b[k]], rp_b[k], sem_g[k]),
                (Cph.at[ci_b[k]], cp_b[k], sem_g[k]),
                (Rgh.at[ri_b[k]], rg_b[k], sem_g[k]),
                (Cgh.at[ci_b[k]], cg_b[k], sem_g[k]),
                (ein_h.at[pl.ds(off, _K)], ein_b[k], sem_g[k]),
                (eout_h.at[pl.ds(off, _K)], eout_b[k], sem_g[k]))

    def sstage(k):
        return ((rp_b[k], acc_s.at[sci_b[k]], sem_s[k]),
                (cp_b[k], acc_s.at[sri_b[k]], sem_s[k]))

    def wstage(j, k):
        off = base + j * _K
        return ((rg_b[k], ab2_h.at[pl.ds(off, _K)], sem_w[k]),)

    def start_all(triples, add=False):
        for src, dst, sem in triples:
            pltpu.async_copy(src, dst, sem, add=add)

    def wait_all(triples):
        for src, dst, sem in triples:
            pltpu.make_async_copy(src, dst, sem).wait()

    def compute(k):
        rp, cp = rp_b[k], cp_b[k]
        rg, cg = rg_b[k], cg_b[k]
        ein, eout = ein_b[k], eout_b[k]
        for h in range(2):
            sri_b[k][pl.ds(h * 16, 16)] = ri_b[k][pl.ds(h * 16, 16)]
            sci_b[k][pl.ds(h * 16, 16)] = ci_b[k][pl.ds(h * 16, 16)]

        def gates(t, carry):
            for u in range(4):
                e = t * 4 + u
                a = rg[e, pl.ds(0, 16)] + ein[e, :] + cg[e, pl.ds(0, 16)]
                b = rg[e, pl.ds(16, 16)] + eout[e, :] + cg[e, pl.ds(16, 16)]
                rg[e, pl.ds(0, 16)] = a
                rg[e, pl.ds(16, 16)] = b
                zi = jnp.broadcast_to(jnp.sum(a * wir, axis=0), (16,))
                zo = jnp.broadcast_to(jnp.sum(b * wor, axis=0), (16,))
                cg[e, pl.ds(0, 16)] = 1.0 / (1.0 + jnp.exp(-zi))
                cg[e, pl.ds(16, 16)] = 1.0 / (1.0 + jnp.exp(-zo))
            return carry

        lax.fori_loop(0, _K // 4, gates, 0)

        def scale(t, carry):
            for u in range(2):
                e = t * 2 + u
                gi = cg[e, pl.ds(0, 16)]
                go = cg[e, pl.ds(16, 16)]
                for q in range(8):
                    rp[e, pl.ds(q * 16, 16)] = gi * rp[e, pl.ds(q * 16, 16)]
                    cp[e, pl.ds(q * 16, 16)] = go * cp[e, pl.ds(q * 16, 16)]
            return carry

        lax.fori_loop(0, _K // 2, scale, 0)

    start_all(istage(0, 0))
    start_all(istage(1, 1))
    start_all(istage(2, 2))
    wait_all(istage(0, 0))
    start_all(gstage(0, 0))
    wait_all(istage(1, 1))
    start_all(gstage(1, 1))

    def ring(g, carry):
        for u in range(_GRP):
            j = g * _GRP + u

            @pl.when(j >= 2)
            def _():
                wait_all(sstage((u + 2) % 4))
                wait_all(wstage(j - 2, (u + 2) % 4))

            @pl.when(j + 3 < _NCH)
            def _():
                start_all(istage(j + 3, (u + 3) % 4))

            @pl.when(j + 2 < _NCH)
            def _():
                wait_all(istage(j + 2, (u + 2) % 4))
                start_all(gstage(j + 2, (u + 2) % 4))

            wait_all(gstage(j, u))
            compute(u)
            start_all(sstage(u), add=True)
            start_all(wstage(j, u))
        return carry

    lax.fori_loop(0, _NCH // _GRP, ring, 0)
    for j in (_NCH - 2, _NCH - 1):
        u = j % _GRP
        wait_all(sstage(u))
        wait_all(wstage(j, u))
    plsc.subcore_barrier()
    pltpu.sync_copy(acc_s.at[pl.ds(s * _RPS, _RPS)],
                    acc_h.at[pl.ds(c * _NPAD + s * _RPS, _RPS)])


def _sc_sparse(Rp, Cp, Rgt, Cgt, ridx, cidx, eawin, eawout, wi1, wo1,
               zeros_nd):
    mesh = plsc.VectorSubcoreMesh(core_axis_name="c", subcore_axis_name="s")
    f = pl.kernel(
        _sc_body,
        out_type=(jax.ShapeDtypeStruct((2 * _NPAD, 128), jnp.float32),
                  jax.ShapeDtypeStruct((_EPAD, 32), jnp.float32)),
        mesh=mesh,
        compiler_params=pltpu.CompilerParams(needs_layout_passes=False,
                                             use_tc_tiling_on_sc=False),
        scratch_types=[
            pltpu.VMEM_SHARED((_NPAD, 128), jnp.float32),
            [pltpu.VMEM((_K, 128), jnp.float32) for _ in range(4)],
            [pltpu.VMEM((_K, 128), jnp.float32) for _ in range(4)],
            [pltpu.VMEM((_K, 32), jnp.float32) for _ in range(4)],
            [pltpu.VMEM((_K, 32), jnp.float32) for _ in range(4)],
            [pltpu.VMEM((_K, 16), jnp.float32) for _ in range(4)],
            [pltpu.VMEM((_K, 16), jnp.float32) for _ in range(4)],
            [pltpu.VMEM((_K,), jnp.int32) for _ in range(4)],
            [pltpu.VMEM((_K,), jnp.int32) for _ in range(4)],
            [pltpu.VMEM((_K,), jnp.int32) for _ in range(4)],
            [pltpu.VMEM((_K,), jnp.int32) for _ in range(4)],
            pltpu.VMEM((16,), jnp.float32),
            pltpu.VMEM((16,), jnp.float32),
            [pltpu.SemaphoreType.DMA for _ in range(4)],
            [pltpu.SemaphoreType.DMA for _ in range(4)],
            [pltpu.SemaphoreType.DMA for _ in range(4)],
            [pltpu.SemaphoreType.DMA for _ in range(4)],
        ],
    )
    return f(Rp, Cp, Rgt, Cgt, ridx, cidx, eawin, eawout, wi1, wo1,
             zeros_nd)


def kernel(x_s, x_t, edge_attr_s, edge_attr_t, edge_index_s, edge_index_t,
           x_s_batch, x_t_batch, y, node_emb, edge_emb, W_in_0, W_in_1,
           W_out_0, W_out_1, gin_W, gin_b, gout_W, gout_b, cls_W1, cls_b1,
           cls_W2, cls_b2):
    zeros_nd = jnp.zeros((_NPAD, 128), jnp.float32)

    def pad_e(a):
        return jnp.concatenate(
            [a, jnp.zeros((_EPAD - _E, 16), jnp.float32)], axis=0)

    def pad_idx(ix):
        return jnp.concatenate(
            [ix.astype(jnp.int32),
             jnp.full((_EPAD - _E,), _N, jnp.int32)], axis=0)

    Tin0 = edge_emb @ W_in_0[0][128:144]
    Tout0 = edge_emb @ W_out_0[0][128:144]

    def dag(x_raw, ea, ridx2, cidx2, batch_ids):
        x0 = jnp.concatenate(
            [_onehot_mm(x_raw, node_emb, 400),
             jnp.zeros((_NPAD - _N, 128), jnp.float32)], axis=0)
        eawin = pad_e(_onehot_mm(ea[:_E], Tin0, 1000))
        eawout = pad_e(_onehot_mm(ea[_E:], Tout0, 1000))
        x = x0
        acc2 = bias = ab2 = None
        z16 = jnp.zeros((16, 16), jnp.float32)
        for i in range(_L):
            W = jnp.concatenate(
                [gin_W[i], gout_W[i], W_in_0[i][:128], W_out_0[i][144:],
                 W_in_0[i][144:], W_out_0[i][:128]], axis=1)
            if i == 0:
                Rp, Cp, Rgt, Cgt = _proj0(x, W)
            else:
                x, Rp, Cp, Rgt, Cgt = _proj(x, acc2, bias, W)
                eawin = _matmul(
                    ab2, jnp.concatenate([W_in_0[i][128:144], z16], axis=0),
                    1024)
                eawout = _matmul(
                    ab2, jnp.concatenate([z16, W_out_0[i][128:144]], axis=0),
                    1024)
            acc2, ab2 = _sc_sparse(
                Rp, Cp, Rgt, Cgt, ridx2, cidx2, eawin, eawout,
                W_in_1[i].reshape(16), W_out_1[i].reshape(16), zeros_nd)
            bias = gin_b[i] + gout_b[i]
        return _pool(x, acc2, bias, batch_ids)

    g_s = dag(x_s, edge_attr_s, pad_idx(edge_index_s[0]),
              pad_idx(edge_index_s[1]), x_s_batch)
    g_t = dag(x_t, edge_attr_t, pad_idx(edge_index_t[0]),
              pad_idx(edge_index_t[1]), x_t_batch)

    B = y.shape[0]
    z = jnp.concatenate([g_s, g_t], axis=1)
    h = jax.nn.relu(z @ cls_W1 + cls_b1)
    pred = h @ cls_W2 + cls_b2
    logp = jax.nn.log_softmax(pred, axis=1)
    loss = -jnp.mean(logp[jnp.arange(B), y])
    return loss
